# baseline (speedup 1.0000x reference)
import functools

import jax
import jax.numpy as jnp
from jax import lax
from jax.experimental import pallas as pl
from jax.experimental.pallas import tpu as pltpu
from jax.experimental.pallas import tpu_sc as plsc

N_PARTICLES = 4194304
N_PIXELS = 8388608
C = 4
BLK = 16
N_BLOCKS = N_PARTICLES * C // BLK
TILE = 128
N_TTILES = N_PARTICLES // TILE
N_OTILES = N_PIXELS // TILE
TW = TILE * C

NUM_CORES = 2
NUM_SUBCORES = 16
NW = NUM_CORES * NUM_SUBCORES
L = 16

T_STEP = 32
R_STEPS = N_TTILES // (NW * T_STEP)
RM_ROWS = T_STEP * TILE // C

B_PER_W = N_PIXELS // NW
IDX_W = 128
K = 8
CHUNK = K * IDX_W
STEPS = B_PER_W // CHUNK
NBUF = 4


def _repack_kernel(tab_hbm, rm_hbm, in0, in1, rm0, rm1, isem0, isem1,
                   osem0, osem1):
    wid = lax.axis_index("s") * NUM_CORES + lax.axis_index("c")
    t_base = wid * (N_TTILES // NW)

    lanes = lax.iota(jnp.int32, L)
    perm = (lanes & 3) * TILE + lax.shift_right_logical(lanes, 2)

    ins = (in0, in1)
    rms = (rm0, rm1)
    isems = (isem0, isem1)
    osems = (osem0, osem1)

    def t0_of(s):
        return pl.multiple_of(t_base + s * T_STEP, T_STEP)

    def fire_in(s, b):
        pltpu.async_copy(
            tab_hbm.at[pl.ds(t0_of(s) * TW, T_STEP * TW)], ins[b], isems[b]
        )

    def drain_in(b):
        pltpu.make_async_copy(
            tab_hbm.at[pl.ds(0, T_STEP * TW)], ins[b], isems[b]
        ).wait()

    def fire_out(s, b):
        pltpu.async_copy(
            rms[b], rm_hbm.at[pl.ds(t0_of(s) * (TILE // C), RM_ROWS)], osems[b]
        )

    def drain_out(b):
        pltpu.make_async_copy(
            rms[b], rm_hbm.at[pl.ds(0, RM_ROWS)], osems[b]
        ).wait()

    def permute(b):
        in_v, rm_v = ins[b], rms[b]

        def tile_body(t, carry):
            for q in range(0, TW, L):
                src = perm + (t * TW + (q >> 2))
                g = plsc.load_gather(in_v, [src])
                rm_v[t * (TW // L) + (q // L), :] = g
            return carry

        lax.fori_loop(0, T_STEP, tile_body, 0)

    fire_in(0, 0)

    def loop_body(h, carry):
        s0 = h * 2
        fire_in(s0 + 1, 1)
        drain_in(0)

        @pl.when(h > 0)
        def _():
            drain_out(0)

        permute(0)
        fire_out(s0, 0)

        @pl.when(h < R_STEPS // 2 - 1)
        def _():
            fire_in(s0 + 2, 0)

        drain_in(1)

        @pl.when(h > 0)
        def _():
            drain_out(1)

        permute(1)
        fire_out(s0 + 1, 1)
        return carry

    lax.fori_loop(0, R_STEPS // 2, loop_body, 0)
    drain_out(0)
    drain_out(1)


def _gather_kernel(rm_hbm, idx_hbm, blk_hbm, out_hbm, *refs):
    idxs = refs[0:NBUF]
    blks = refs[NBUF:2 * NBUF]
    rows = refs[2 * NBUF:3 * NBUF]
    outs = refs[3 * NBUF:4 * NBUF]
    isems = refs[4 * NBUF:5 * NBUF]
    gsems = refs[5 * NBUF:6 * NBUF]
    osems = refs[6 * NBUF:7 * NBUF]

    wid = lax.axis_index("s") * NUM_CORES + lax.axis_index("c")
    base = wid * B_PER_W

    lanes = lax.iota(jnp.int32, L)

    def start_of(s):
        return pl.multiple_of(base + s * CHUNK, CHUNK)

    def fire_idxblk(s, b):
        st = start_of(s)
        pltpu.async_copy(idx_hbm.at[pl.ds(st, CHUNK)], idxs[b], isems[b])
        pltpu.async_copy(blk_hbm.at[pl.ds(st, CHUNK)], blks[b], isems[b])

    def fire_gathers(b):
        pltpu.make_async_copy(idx_hbm.at[pl.ds(0, CHUNK)], idxs[b],
                              isems[b]).wait()
        pltpu.make_async_copy(blk_hbm.at[pl.ds(0, CHUNK)], blks[b],
                              isems[b]).wait()
        for j in range(K):
            pltpu.async_copy(
                rm_hbm.at[blks[b].at[pl.ds(j * IDX_W, IDX_W)]],
                rows[b].at[pl.ds(j * IDX_W, IDX_W)],
                gsems[b],
            )

    def drain_gather(b):
        pltpu.make_async_copy(
            rm_hbm.at[pl.ds(0, CHUNK)], rows[b], gsems[b]
        ).wait()

    def select(b):
        idx_v, rows_v, out_v = idxs[b], rows[b], outs[b]

        def sel_body(k2, carry):
            for u in range(2):
                k = k2 * 2 + u
                off = pl.multiple_of(k * L, L)
                rr = lanes + off
                v = idx_v[pl.ds(off, L)]
                col0 = lax.shift_left(lax.bitwise_and(v, 3), 2)
                tt = k >> 3
                l0 = (k & 7) * L
                for c in range(C):
                    g = plsc.load_gather(rows_v, [rr, col0 + c])
                    out_v[pl.ds(tt * (TILE * C) + c * TILE + l0, L)] = g
            return carry

        lax.fori_loop(0, CHUNK // L // 2, sel_body, 0)

    def fire_out(s, b):
        pltpu.async_copy(
            outs[b], out_hbm.at[pl.ds(start_of(s) * C, CHUNK * C)], osems[b]
        )

    def drain_out(b):
        pltpu.make_async_copy(
            outs[b], out_hbm.at[pl.ds(0, CHUNK * C)], osems[b]
        ).wait()

    for s in range(NBUF - 1):
        fire_idxblk(s, s)
        fire_gathers(s)

    def loop_body(h, carry):
        for u in range(NBUF):
            s = h * NBUF + u
            b = u
            b_next = (u + NBUF - 1) % NBUF

            @pl.when(s + NBUF - 1 < STEPS)
            def _():
                fire_idxblk(s + NBUF - 1, b_next)
                fire_gathers(b_next)

            drain_gather(b)

            @pl.when(h > 0)
            def _():
                drain_out(b)

            select(b)
            fire_out(s, b)
        return carry

    lax.fori_loop(0, STEPS // NBUF, loop_body, 0)
    for b in range(NBUF):
        drain_out(b)


def kernel(input_features, pixel_to_particle):
    idx = pixel_to_particle.astype(jnp.int32)
    blk = lax.shift_right_logical(idx, 2)
    tab_native = (
        input_features.reshape(N_TTILES, TILE, C)
        .transpose(0, 2, 1)
        .reshape(N_PARTICLES * C)
    )
    mesh = plsc.VectorSubcoreMesh(core_axis_name="c", subcore_axis_name="s")
    params = pltpu.CompilerParams(
        use_tc_tiling_on_sc=False, needs_layout_passes=False
    )

    repack = functools.partial(
        pl.kernel,
        mesh=mesh,
        compiler_params=params,
        out_type=jax.ShapeDtypeStruct((N_BLOCKS, BLK), jnp.float32),
        scratch_types=[
            pltpu.VMEM((T_STEP * TW,), jnp.float32),
            pltpu.VMEM((T_STEP * TW,), jnp.float32),
            pltpu.VMEM((RM_ROWS, BLK), jnp.float32),
            pltpu.VMEM((RM_ROWS, BLK), jnp.float32),
            pltpu.SemaphoreType.DMA,
            pltpu.SemaphoreType.DMA,
            pltpu.SemaphoreType.DMA,
            pltpu.SemaphoreType.DMA,
        ],
    )(_repack_kernel)
    rm = repack(tab_native)

    gather = functools.partial(
        pl.kernel,
        mesh=mesh,
        compiler_params=params,
        out_type=jax.ShapeDtypeStruct((N_PIXELS * C,), jnp.float32),
        scratch_types=(
            [pltpu.VMEM((CHUNK,), jnp.int32)] * NBUF
            + [pltpu.VMEM((CHUNK,), jnp.int32)] * NBUF
            + [pltpu.VMEM((CHUNK, BLK), jnp.float32)] * NBUF
            + [pltpu.VMEM((CHUNK * C,), jnp.float32)] * NBUF
            + [pltpu.SemaphoreType.DMA] * (3 * NBUF)
        ),
    )(_gather_kernel)
    out_native = gather(rm, idx, blk)

    return (
        out_native.reshape(N_OTILES, C, TILE)
        .transpose(0, 2, 1)
        .reshape(N_PIXELS, C)
    )


# speedup vs baseline: 1.1081x; 1.1081x over previous
import functools

import jax
import jax.numpy as jnp
from jax import lax
from jax.experimental import pallas as pl
from jax.experimental.pallas import tpu as pltpu
from jax.experimental.pallas import tpu_sc as plsc

N_PARTICLES = 4194304
N_PIXELS = 8388608
C = 4
BLK = 16
N_BLOCKS = N_PARTICLES * C // BLK
TILE = 128
N_TTILES = N_PARTICLES // TILE
N_OTILES = N_PIXELS // TILE
TW = TILE * C

NUM_CORES = 2
NUM_SUBCORES = 16
NW = NUM_CORES * NUM_SUBCORES
L = 16

T_STEP = 32
R_STEPS = N_TTILES // (NW * T_STEP)
RM_ROWS = T_STEP * TILE // C

B_PER_W = N_PIXELS // NW
IDX_W = 256
K = 8
CHUNK = K * IDX_W
STEPS = B_PER_W // CHUNK
NBUF = 2


def _repack_kernel(tab_hbm, rm_hbm, in0, in1, rm0, rm1, isem0, isem1,
                   osem0, osem1):
    wid = lax.axis_index("s") * NUM_CORES + lax.axis_index("c")
    t_base = wid * (N_TTILES // NW)

    lanes = lax.iota(jnp.int32, L)
    perm = (lanes & 3) * TILE + lax.shift_right_logical(lanes, 2)

    ins = (in0, in1)
    rms = (rm0, rm1)
    isems = (isem0, isem1)
    osems = (osem0, osem1)

    def t0_of(s):
        return pl.multiple_of(t_base + s * T_STEP, T_STEP)

    def fire_in(s, b):
        pltpu.async_copy(
            tab_hbm.at[pl.ds(t0_of(s) * TW, T_STEP * TW)], ins[b], isems[b]
        )

    def drain_in(b):
        pltpu.make_async_copy(
            tab_hbm.at[pl.ds(0, T_STEP * TW)], ins[b], isems[b]
        ).wait()

    def fire_out(s, b):
        pltpu.async_copy(
            rms[b], rm_hbm.at[pl.ds(t0_of(s) * (TILE // C), RM_ROWS)], osems[b]
        )

    def drain_out(b):
        pltpu.make_async_copy(
            rms[b], rm_hbm.at[pl.ds(0, RM_ROWS)], osems[b]
        ).wait()

    def permute(b):
        in_v, rm_v = ins[b], rms[b]

        def tile_body(t, carry):
            for q in range(0, TW, L):
                src = perm + (t * TW + (q >> 2))
                g = plsc.load_gather(in_v, [src])
                rm_v[t * (TW // L) + (q // L), :] = g
            return carry

        lax.fori_loop(0, T_STEP, tile_body, 0)

    fire_in(0, 0)

    def loop_body(h, carry):
        s0 = h * 2
        fire_in(s0 + 1, 1)
        drain_in(0)

        @pl.when(h > 0)
        def _():
            drain_out(0)

        permute(0)
        fire_out(s0, 0)

        @pl.when(h < R_STEPS // 2 - 1)
        def _():
            fire_in(s0 + 2, 0)

        drain_in(1)

        @pl.when(h > 0)
        def _():
            drain_out(1)

        permute(1)
        fire_out(s0 + 1, 1)
        return carry

    lax.fori_loop(0, R_STEPS // 2, loop_body, 0)
    drain_out(0)
    drain_out(1)


def _gather_kernel(rm_hbm, idx_hbm, blk_hbm, out_hbm, *refs):
    idxs = refs[0:NBUF]
    blks = refs[NBUF:2 * NBUF]
    rows = refs[2 * NBUF:3 * NBUF]
    outs = refs[3 * NBUF:4 * NBUF]
    isems = refs[4 * NBUF:5 * NBUF]
    gsems = refs[5 * NBUF:6 * NBUF]
    osems = refs[6 * NBUF:7 * NBUF]

    wid = lax.axis_index("s") * NUM_CORES + lax.axis_index("c")
    base = wid * B_PER_W

    lanes = lax.iota(jnp.int32, L)

    def start_of(s):
        return pl.multiple_of(base + s * CHUNK, CHUNK)

    def fire_idxblk(s, b):
        st = start_of(s)
        pltpu.async_copy(idx_hbm.at[pl.ds(st, CHUNK)], idxs[b], isems[b])
        pltpu.async_copy(blk_hbm.at[pl.ds(st, CHUNK)], blks[b], isems[b])

    def fire_gathers(b):
        pltpu.make_async_copy(idx_hbm.at[pl.ds(0, CHUNK)], idxs[b],
                              isems[b]).wait()
        pltpu.make_async_copy(blk_hbm.at[pl.ds(0, CHUNK)], blks[b],
                              isems[b]).wait()
        for j in range(K):
            pltpu.async_copy(
                rm_hbm.at[blks[b].at[pl.ds(j * IDX_W, IDX_W)]],
                rows[b].at[pl.ds(j * IDX_W, IDX_W)],
                gsems[b],
            )

    def drain_gather(b):
        pltpu.make_async_copy(
            rm_hbm.at[pl.ds(0, CHUNK)], rows[b], gsems[b]
        ).wait()

    def select(b):
        idx_v, rows_v, out_v = idxs[b], rows[b], outs[b]

        def sel_body(k2, carry):
            for u in range(2):
                k = k2 * 2 + u
                off = pl.multiple_of(k * L, L)
                rr = lanes + off
                v = idx_v[pl.ds(off, L)]
                col0 = lax.shift_left(lax.bitwise_and(v, 3), 2)
                tt = k >> 3
                l0 = (k & 7) * L
                for c in range(C):
                    g = plsc.load_gather(rows_v, [rr, col0 + c])
                    out_v[pl.ds(tt * (TILE * C) + c * TILE + l0, L)] = g
            return carry

        lax.fori_loop(0, CHUNK // L // 2, sel_body, 0)

    def fire_out(s, b):
        pltpu.async_copy(
            outs[b], out_hbm.at[pl.ds(start_of(s) * C, CHUNK * C)], osems[b]
        )

    def drain_out(b):
        pltpu.make_async_copy(
            outs[b], out_hbm.at[pl.ds(0, CHUNK * C)], osems[b]
        ).wait()

    for s in range(NBUF - 1):
        fire_idxblk(s, s)
        fire_gathers(s)

    def loop_body(h, carry):
        for u in range(NBUF):
            s = h * NBUF + u
            b = u
            b_next = (u + NBUF - 1) % NBUF

            @pl.when(s + NBUF - 1 < STEPS)
            def _():
                fire_idxblk(s + NBUF - 1, b_next)
                fire_gathers(b_next)

            drain_gather(b)

            @pl.when(h > 0)
            def _():
                drain_out(b)

            select(b)
            fire_out(s, b)
        return carry

    lax.fori_loop(0, STEPS // NBUF, loop_body, 0)
    for b in range(NBUF):
        drain_out(b)


def kernel(input_features, pixel_to_particle):
    idx = pixel_to_particle.astype(jnp.int32)
    blk = lax.shift_right_logical(idx, 2)
    tab_native = (
        input_features.reshape(N_TTILES, TILE, C)
        .transpose(0, 2, 1)
        .reshape(N_PARTICLES * C)
    )
    mesh = plsc.VectorSubcoreMesh(core_axis_name="c", subcore_axis_name="s")
    params = pltpu.CompilerParams(
        use_tc_tiling_on_sc=False, needs_layout_passes=False
    )

    repack = functools.partial(
        pl.kernel,
        mesh=mesh,
        compiler_params=params,
        out_type=jax.ShapeDtypeStruct((N_BLOCKS, BLK), jnp.float32),
        scratch_types=[
            pltpu.VMEM((T_STEP * TW,), jnp.float32),
            pltpu.VMEM((T_STEP * TW,), jnp.float32),
            pltpu.VMEM((RM_ROWS, BLK), jnp.float32),
            pltpu.VMEM((RM_ROWS, BLK), jnp.float32),
            pltpu.SemaphoreType.DMA,
            pltpu.SemaphoreType.DMA,
            pltpu.SemaphoreType.DMA,
            pltpu.SemaphoreType.DMA,
        ],
    )(_repack_kernel)
    rm = repack(tab_native)

    gather = functools.partial(
        pl.kernel,
        mesh=mesh,
        compiler_params=params,
        out_type=jax.ShapeDtypeStruct((N_PIXELS * C,), jnp.float32),
        scratch_types=(
            [pltpu.VMEM((CHUNK,), jnp.int32)] * NBUF
            + [pltpu.VMEM((CHUNK,), jnp.int32)] * NBUF
            + [pltpu.VMEM((CHUNK, BLK), jnp.float32)] * NBUF
            + [pltpu.VMEM((CHUNK * C,), jnp.float32)] * NBUF
            + [pltpu.SemaphoreType.DMA] * (3 * NBUF)
        ),
    )(_gather_kernel)
    out_native = gather(rm, idx, blk)

    return (
        out_native.reshape(N_OTILES, C, TILE)
        .transpose(0, 2, 1)
        .reshape(N_PIXELS, C)
    )


# speedup vs baseline: 1.1083x; 1.0002x over previous
import functools

import jax
import jax.numpy as jnp
from jax import lax
from jax.experimental import pallas as pl
from jax.experimental.pallas import tpu as pltpu
from jax.experimental.pallas import tpu_sc as plsc

N_PARTICLES = 4194304
N_PIXELS = 8388608
C = 4
BLK = 16
N_BLOCKS = N_PARTICLES * C // BLK
TILE = 128
N_TTILES = N_PARTICLES // TILE
N_OTILES = N_PIXELS // TILE
TW = TILE * C

NUM_CORES = 2
NUM_SUBCORES = 16
NW = NUM_CORES * NUM_SUBCORES
L = 16

T_STEP = 32
R_STEPS = N_TTILES // (NW * T_STEP)
RM_ROWS = T_STEP * TILE // C

B_PER_W = N_PIXELS // NW
IDX_W = 512
K = 4
CHUNK = K * IDX_W
STEPS = B_PER_W // CHUNK
NBUF = 2


def _repack_kernel(tab_hbm, rm_hbm, in0, in1, rm0, rm1, isem0, isem1,
                   osem0, osem1):
    wid = lax.axis_index("s") * NUM_CORES + lax.axis_index("c")
    t_base = wid * (N_TTILES // NW)

    lanes = lax.iota(jnp.int32, L)
    perm = (lanes & 3) * TILE + lax.shift_right_logical(lanes, 2)

    ins = (in0, in1)
    rms = (rm0, rm1)
    isems = (isem0, isem1)
    osems = (osem0, osem1)

    def t0_of(s):
        return pl.multiple_of(t_base + s * T_STEP, T_STEP)

    def fire_in(s, b):
        pltpu.async_copy(
            tab_hbm.at[pl.ds(t0_of(s) * TW, T_STEP * TW)], ins[b], isems[b]
        )

    def drain_in(b):
        pltpu.make_async_copy(
            tab_hbm.at[pl.ds(0, T_STEP * TW)], ins[b], isems[b]
        ).wait()

    def fire_out(s, b):
        pltpu.async_copy(
            rms[b], rm_hbm.at[pl.ds(t0_of(s) * (TILE // C), RM_ROWS)], osems[b]
        )

    def drain_out(b):
        pltpu.make_async_copy(
            rms[b], rm_hbm.at[pl.ds(0, RM_ROWS)], osems[b]
        ).wait()

    def permute(b):
        in_v, rm_v = ins[b], rms[b]

        def tile_body(t, carry):
            for q in range(0, TW, L):
                src = perm + (t * TW + (q >> 2))
                g = plsc.load_gather(in_v, [src])
                rm_v[t * (TW // L) + (q // L), :] = g
            return carry

        lax.fori_loop(0, T_STEP, tile_body, 0)

    fire_in(0, 0)

    def loop_body(h, carry):
        s0 = h * 2
        fire_in(s0 + 1, 1)
        drain_in(0)

        @pl.when(h > 0)
        def _():
            drain_out(0)

        permute(0)
        fire_out(s0, 0)

        @pl.when(h < R_STEPS // 2 - 1)
        def _():
            fire_in(s0 + 2, 0)

        drain_in(1)

        @pl.when(h > 0)
        def _():
            drain_out(1)

        permute(1)
        fire_out(s0 + 1, 1)
        return carry

    lax.fori_loop(0, R_STEPS // 2, loop_body, 0)
    drain_out(0)
    drain_out(1)


def _gather_kernel(rm_hbm, idx_hbm, blk_hbm, out_hbm, *refs):
    idxs = refs[0:NBUF]
    blks = refs[NBUF:2 * NBUF]
    rows = refs[2 * NBUF:3 * NBUF]
    outs = refs[3 * NBUF:4 * NBUF]
    isems = refs[4 * NBUF:5 * NBUF]
    gsems = refs[5 * NBUF:6 * NBUF]
    osems = refs[6 * NBUF:7 * NBUF]

    wid = lax.axis_index("s") * NUM_CORES + lax.axis_index("c")
    base = wid * B_PER_W

    lanes = lax.iota(jnp.int32, L)

    def start_of(s):
        return pl.multiple_of(base + s * CHUNK, CHUNK)

    def fire_idxblk(s, b):
        st = start_of(s)
        pltpu.async_copy(idx_hbm.at[pl.ds(st, CHUNK)], idxs[b], isems[b])
        pltpu.async_copy(blk_hbm.at[pl.ds(st, CHUNK)], blks[b], isems[b])

    def fire_gathers(b):
        pltpu.make_async_copy(idx_hbm.at[pl.ds(0, CHUNK)], idxs[b],
                              isems[b]).wait()
        pltpu.make_async_copy(blk_hbm.at[pl.ds(0, CHUNK)], blks[b],
                              isems[b]).wait()
        for j in range(K):
            pltpu.async_copy(
                rm_hbm.at[blks[b].at[pl.ds(j * IDX_W, IDX_W)]],
                rows[b].at[pl.ds(j * IDX_W, IDX_W)],
                gsems[b],
            )

    def drain_gather(b):
        pltpu.make_async_copy(
            rm_hbm.at[pl.ds(0, CHUNK)], rows[b], gsems[b]
        ).wait()

    def select(b):
        idx_v, rows_v, out_v = idxs[b], rows[b], outs[b]

        def sel_body(k2, carry):
            for u in range(2):
                k = k2 * 2 + u
                off = pl.multiple_of(k * L, L)
                rr = lanes + off
                v = idx_v[pl.ds(off, L)]
                col0 = lax.shift_left(lax.bitwise_and(v, 3), 2)
                tt = k >> 3
                l0 = (k & 7) * L
                for c in range(C):
                    g = plsc.load_gather(rows_v, [rr, col0 + c])
                    out_v[pl.ds(tt * (TILE * C) + c * TILE + l0, L)] = g
            return carry

        lax.fori_loop(0, CHUNK // L // 2, sel_body, 0)

    def fire_out(s, b):
        pltpu.async_copy(
            outs[b], out_hbm.at[pl.ds(start_of(s) * C, CHUNK * C)], osems[b]
        )

    def drain_out(b):
        pltpu.make_async_copy(
            outs[b], out_hbm.at[pl.ds(0, CHUNK * C)], osems[b]
        ).wait()

    for s in range(NBUF - 1):
        fire_idxblk(s, s)
        fire_gathers(s)

    def loop_body(h, carry):
        for u in range(NBUF):
            s = h * NBUF + u
            b = u
            b_next = (u + NBUF - 1) % NBUF

            @pl.when(s + NBUF - 1 < STEPS)
            def _():
                fire_idxblk(s + NBUF - 1, b_next)
                fire_gathers(b_next)

            drain_gather(b)

            @pl.when(h > 0)
            def _():
                drain_out(b)

            select(b)
            fire_out(s, b)
        return carry

    lax.fori_loop(0, STEPS // NBUF, loop_body, 0)
    for b in range(NBUF):
        drain_out(b)


def kernel(input_features, pixel_to_particle):
    idx = pixel_to_particle.astype(jnp.int32)
    blk = lax.shift_right_logical(idx, 2)
    tab_native = (
        input_features.reshape(N_TTILES, TILE, C)
        .transpose(0, 2, 1)
        .reshape(N_PARTICLES * C)
    )
    mesh = plsc.VectorSubcoreMesh(core_axis_name="c", subcore_axis_name="s")
    params = pltpu.CompilerParams(
        use_tc_tiling_on_sc=False, needs_layout_passes=False
    )

    repack = functools.partial(
        pl.kernel,
        mesh=mesh,
        compiler_params=params,
        out_type=jax.ShapeDtypeStruct((N_BLOCKS, BLK), jnp.float32),
        scratch_types=[
            pltpu.VMEM((T_STEP * TW,), jnp.float32),
            pltpu.VMEM((T_STEP * TW,), jnp.float32),
            pltpu.VMEM((RM_ROWS, BLK), jnp.float32),
            pltpu.VMEM((RM_ROWS, BLK), jnp.float32),
            pltpu.SemaphoreType.DMA,
            pltpu.SemaphoreType.DMA,
            pltpu.SemaphoreType.DMA,
            pltpu.SemaphoreType.DMA,
        ],
    )(_repack_kernel)
    rm = repack(tab_native)

    gather = functools.partial(
        pl.kernel,
        mesh=mesh,
        compiler_params=params,
        out_type=jax.ShapeDtypeStruct((N_PIXELS * C,), jnp.float32),
        scratch_types=(
            [pltpu.VMEM((CHUNK,), jnp.int32)] * NBUF
            + [pltpu.VMEM((CHUNK,), jnp.int32)] * NBUF
            + [pltpu.VMEM((CHUNK, BLK), jnp.float32)] * NBUF
            + [pltpu.VMEM((CHUNK * C,), jnp.float32)] * NBUF
            + [pltpu.SemaphoreType.DMA] * (3 * NBUF)
        ),
    )(_gather_kernel)
    out_native = gather(rm, idx, blk)

    return (
        out_native.reshape(N_OTILES, C, TILE)
        .transpose(0, 2, 1)
        .reshape(N_PIXELS, C)
    )


# speedup vs baseline: 1.1315x; 1.0209x over previous
import functools

import jax
import jax.numpy as jnp
from jax import lax
from jax.experimental import pallas as pl
from jax.experimental.pallas import tpu as pltpu
from jax.experimental.pallas import tpu_sc as plsc

N_PARTICLES = 4194304
N_PIXELS = 8388608
C = 4
BLK = 16
N_BLOCKS = N_PARTICLES * C // BLK
GBLK = 8
N_GBLOCKS = N_PARTICLES * C // GBLK
TILE = 128
N_TTILES = N_PARTICLES // TILE
N_OTILES = N_PIXELS // TILE
TW = TILE * C

NUM_CORES = 2
NUM_SUBCORES = 16
NW = NUM_CORES * NUM_SUBCORES
L = 16

T_STEP = 32
R_STEPS = N_TTILES // (NW * T_STEP)
RM_ROWS = T_STEP * TILE // C

B_PER_W = N_PIXELS // NW
IDX_W = 512
K = 4
CHUNK = K * IDX_W
STEPS = B_PER_W // CHUNK
NBUF = 2


def _repack_kernel(tab_hbm, rm_hbm, in0, in1, rm0, rm1, isem0, isem1,
                   osem0, osem1):
    wid = lax.axis_index("s") * NUM_CORES + lax.axis_index("c")
    t_base = wid * (N_TTILES // NW)

    lanes = lax.iota(jnp.int32, L)
    perm = (lanes & 3) * TILE + lax.shift_right_logical(lanes, 2)

    ins = (in0, in1)
    rms = (rm0, rm1)
    isems = (isem0, isem1)
    osems = (osem0, osem1)

    def t0_of(s):
        return pl.multiple_of(t_base + s * T_STEP, T_STEP)

    def fire_in(s, b):
        pltpu.async_copy(
            tab_hbm.at[pl.ds(t0_of(s) * TW, T_STEP * TW)], ins[b], isems[b]
        )

    def drain_in(b):
        pltpu.make_async_copy(
            tab_hbm.at[pl.ds(0, T_STEP * TW)], ins[b], isems[b]
        ).wait()

    def fire_out(s, b):
        pltpu.async_copy(
            rms[b], rm_hbm.at[pl.ds(t0_of(s) * (TILE // C), RM_ROWS)], osems[b]
        )

    def drain_out(b):
        pltpu.make_async_copy(
            rms[b], rm_hbm.at[pl.ds(0, RM_ROWS)], osems[b]
        ).wait()

    def permute(b):
        in_v, rm_v = ins[b], rms[b]

        def tile_body(t, carry):
            for q in range(0, TW, L):
                src = perm + (t * TW + (q >> 2))
                g = plsc.load_gather(in_v, [src])
                rm_v[t * (TW // L) + (q // L), :] = g
            return carry

        lax.fori_loop(0, T_STEP, tile_body, 0)

    fire_in(0, 0)

    def loop_body(h, carry):
        s0 = h * 2
        fire_in(s0 + 1, 1)
        drain_in(0)

        @pl.when(h > 0)
        def _():
            drain_out(0)

        permute(0)
        fire_out(s0, 0)

        @pl.when(h < R_STEPS // 2 - 1)
        def _():
            fire_in(s0 + 2, 0)

        drain_in(1)

        @pl.when(h > 0)
        def _():
            drain_out(1)

        permute(1)
        fire_out(s0 + 1, 1)
        return carry

    lax.fori_loop(0, R_STEPS // 2, loop_body, 0)
    drain_out(0)
    drain_out(1)


def _gather_kernel(rm_hbm, idx_hbm, blk_hbm, out_hbm, *refs):
    idxs = refs[0:NBUF]
    blks = refs[NBUF:2 * NBUF]
    rows = refs[2 * NBUF:3 * NBUF]
    outs = refs[3 * NBUF:4 * NBUF]
    isems = refs[4 * NBUF:5 * NBUF]
    gsems = refs[5 * NBUF:6 * NBUF]
    osems = refs[6 * NBUF:7 * NBUF]

    wid = lax.axis_index("s") * NUM_CORES + lax.axis_index("c")
    base = wid * B_PER_W

    lanes = lax.iota(jnp.int32, L)

    def start_of(s):
        return pl.multiple_of(base + s * CHUNK, CHUNK)

    def fire_idxblk(s, b):
        st = start_of(s)
        pltpu.async_copy(idx_hbm.at[pl.ds(st, CHUNK)], idxs[b], isems[b])
        pltpu.async_copy(blk_hbm.at[pl.ds(st, CHUNK)], blks[b], isems[b])

    def fire_gathers(b):
        pltpu.make_async_copy(idx_hbm.at[pl.ds(0, CHUNK)], idxs[b],
                              isems[b]).wait()
        pltpu.make_async_copy(blk_hbm.at[pl.ds(0, CHUNK)], blks[b],
                              isems[b]).wait()
        for j in range(K):
            pltpu.async_copy(
                rm_hbm.at[blks[b].at[pl.ds(j * IDX_W, IDX_W)]],
                rows[b].at[pl.ds(j * IDX_W, IDX_W)],
                gsems[b],
            )

    def drain_gather(b):
        pltpu.make_async_copy(
            rm_hbm.at[pl.ds(0, CHUNK)], rows[b], gsems[b]
        ).wait()

    def select(b):
        idx_v, rows_v, out_v = idxs[b], rows[b], outs[b]

        def sel_body(k2, carry):
            for u in range(2):
                k = k2 * 2 + u
                off = pl.multiple_of(k * L, L)
                rr = lanes + off
                v = idx_v[pl.ds(off, L)]
                col0 = lax.shift_left(lax.bitwise_and(v, 1), 2)
                tt = k >> 3
                l0 = (k & 7) * L
                for c in range(C):
                    g = plsc.load_gather(rows_v, [rr, col0 + c])
                    out_v[pl.ds(tt * (TILE * C) + c * TILE + l0, L)] = g
            return carry

        lax.fori_loop(0, CHUNK // L // 2, sel_body, 0)

    def fire_out(s, b):
        pltpu.async_copy(
            outs[b], out_hbm.at[pl.ds(start_of(s) * C, CHUNK * C)], osems[b]
        )

    def drain_out(b):
        pltpu.make_async_copy(
            outs[b], out_hbm.at[pl.ds(0, CHUNK * C)], osems[b]
        ).wait()

    for s in range(NBUF - 1):
        fire_idxblk(s, s)
        fire_gathers(s)

    def loop_body(h, carry):
        for u in range(NBUF):
            s = h * NBUF + u
            b = u
            b_next = (u + NBUF - 1) % NBUF

            @pl.when(s + NBUF - 1 < STEPS)
            def _():
                fire_idxblk(s + NBUF - 1, b_next)
                fire_gathers(b_next)

            drain_gather(b)

            @pl.when(h > 0)
            def _():
                drain_out(b)

            select(b)
            fire_out(s, b)
        return carry

    lax.fori_loop(0, STEPS // NBUF, loop_body, 0)
    for b in range(NBUF):
        drain_out(b)


def kernel(input_features, pixel_to_particle):
    idx = pixel_to_particle.astype(jnp.int32)
    blk = lax.shift_right_logical(idx, 1)
    tab_native = (
        input_features.reshape(N_TTILES, TILE, C)
        .transpose(0, 2, 1)
        .reshape(N_PARTICLES * C)
    )
    mesh = plsc.VectorSubcoreMesh(core_axis_name="c", subcore_axis_name="s")
    params = pltpu.CompilerParams(
        use_tc_tiling_on_sc=False, needs_layout_passes=False
    )

    repack = functools.partial(
        pl.kernel,
        mesh=mesh,
        compiler_params=params,
        out_type=jax.ShapeDtypeStruct((N_BLOCKS, BLK), jnp.float32),
        scratch_types=[
            pltpu.VMEM((T_STEP * TW,), jnp.float32),
            pltpu.VMEM((T_STEP * TW,), jnp.float32),
            pltpu.VMEM((RM_ROWS, BLK), jnp.float32),
            pltpu.VMEM((RM_ROWS, BLK), jnp.float32),
            pltpu.SemaphoreType.DMA,
            pltpu.SemaphoreType.DMA,
            pltpu.SemaphoreType.DMA,
            pltpu.SemaphoreType.DMA,
        ],
    )(_repack_kernel)
    rm = repack(tab_native)

    gather = functools.partial(
        pl.kernel,
        mesh=mesh,
        compiler_params=params,
        out_type=jax.ShapeDtypeStruct((N_PIXELS * C,), jnp.float32),
        scratch_types=(
            [pltpu.VMEM((CHUNK,), jnp.int32)] * NBUF
            + [pltpu.VMEM((CHUNK,), jnp.int32)] * NBUF
            + [pltpu.VMEM((CHUNK, GBLK), jnp.float32)] * NBUF
            + [pltpu.VMEM((CHUNK * C,), jnp.float32)] * NBUF
            + [pltpu.SemaphoreType.DMA] * (3 * NBUF)
        ),
    )(_gather_kernel)
    out_native = gather(rm.reshape(N_GBLOCKS, GBLK), idx, blk)

    return (
        out_native.reshape(N_OTILES, C, TILE)
        .transpose(0, 2, 1)
        .reshape(N_PIXELS, C)
    )
